# baseline (speedup 1.0000x reference)
import functools

import jax
import jax.numpy as jnp
from jax import lax
from jax.experimental import pallas as pl
from jax.experimental.pallas import tpu as pltpu
from jax.experimental.pallas import tpu_sc as plsc

N = 10000
E = 320000
D = 128

NC = 2
NS = 16
NW = NC * NS
EPW = E // NW
K = 80
CH = EPW // K
KA = 40
CHA = EPW // KA
CHD = EPW // KA
RING = 6
TRB = 624
LAST0 = TRB * (NS - 1)
LASTR = N - LAST0


def _make_agg_kernel(with_deg: bool):
    mesh = plsc.VectorSubcoreMesh(core_axis_name="c", subcore_axis_name="s")
    scratch = (
        [pltpu.VMEM((RING, KA), jnp.int32)] +
        [pltpu.VMEM((RING, KA), jnp.int32)] +
        [pltpu.VMEM((RING, KA, D), jnp.float32)] +
        [pltpu.VMEM_SHARED((N, D), jnp.float32)] +
        [pltpu.SemaphoreType.DMA] * (3 * RING)
    )
    if with_deg:
        scratch = scratch + [
            pltpu.VMEM((K, D), jnp.float32),
            pltpu.VMEM((2, K), jnp.int32),
        ]

    def body(h_hbm, srcr_hbm, dstr_hbm, zrows_hbm, *rest):
        if with_deg:
            (ones_hbm, agg_out, deg_out, src_v, dst_v, rows_v, agg_sh,
             *sems) = rest
            ones_v, ddst_v = sems[-2], sems[-1]
            sems = sems[:-2]
        else:
            agg_out, src_v, dst_v, rows_v, agg_sh, *sems = rest
        si = sems[0:RING]
        sg = sems[RING:2 * RING]
        ss = sems[2 * RING:3 * RING]
        c = lax.axis_index("c")
        s = lax.axis_index("s")
        wid = s * NC + c
        row0 = pl.multiple_of(s * TRB, 8)

        def zero_acc():
            @pl.when(s < NS - 1)
            def _():
                pltpu.sync_copy(zrows_hbm.at[pl.ds(0, TRB)],
                                agg_sh.at[pl.ds(row0, TRB)])

            @pl.when(s == NS - 1)
            def _():
                pltpu.sync_copy(zrows_hbm.at[pl.ds(0, LASTR)],
                                agg_sh.at[pl.ds(LAST0, LASTR)])

        zero_acc()
        plsc.subcore_barrier()

        base = wid * EPW

        def fire_idx(b, chunk):
            off = pl.multiple_of(base + chunk * KA, 8)
            pltpu.async_copy(srcr_hbm.at[pl.ds(off, KA)], src_v.at[b], si[b])
            pltpu.async_copy(dstr_hbm.at[pl.ds(off, KA)], dst_v.at[b], si[b])

        def wait_idx(b):
            pltpu.make_async_copy(srcr_hbm.at[pl.ds(0, KA)], src_v.at[b],
                                  si[b]).wait()
            pltpu.make_async_copy(dstr_hbm.at[pl.ds(0, KA)], dst_v.at[b],
                                  si[b]).wait()

        HKA = 24
        HKB = KA - HKA

        def fire_gather(b):
            pltpu.async_copy(h_hbm.at[src_v.at[b].at[pl.ds(0, HKA)]],
                             rows_v.at[b].at[pl.ds(0, HKA)], sg[b])
            pltpu.async_copy(h_hbm.at[src_v.at[b].at[pl.ds(HKA, HKB)]],
                             rows_v.at[b].at[pl.ds(HKA, HKB)], sg[b])

        def wait_gather(b):
            pltpu.make_async_copy(h_hbm.at[src_v.at[b].at[pl.ds(0, HKA)]],
                                  rows_v.at[b].at[pl.ds(0, HKA)],
                                  sg[b]).wait()
            pltpu.make_async_copy(h_hbm.at[src_v.at[b].at[pl.ds(HKA, HKB)]],
                                  rows_v.at[b].at[pl.ds(HKA, HKB)],
                                  sg[b]).wait()

        def fire_scatter(b):
            pltpu.async_copy(rows_v.at[b], agg_sh.at[dst_v.at[b]], ss[b],
                             add=True)

        def wait_scatter(b):
            pltpu.make_async_copy(rows_v.at[b], agg_sh.at[dst_v.at[b]],
                                  ss[b]).wait()

        def step(i, carry):
            for u in range(RING):
                t = RING * i + u
                @pl.when(jnp.logical_and(t >= 5, t < CHA + 5))
                def _(u=u):
                    wait_scatter((u - 5) % RING)

                @pl.when(jnp.logical_and(t >= 4, t < CHA + 4))
                def _(u=u):
                    wait_gather((u - 4) % RING)
                    fire_scatter((u - 4) % RING)

                @pl.when(jnp.logical_and(t >= 1, t < CHA + 1))
                def _(u=u):
                    wait_idx((u - 1) % RING)
                    fire_gather((u - 1) % RING)

                @pl.when(t < CHA)
                def _(u=u, t=t):
                    fire_idx(u, t)
            return carry

        lax.fori_loop(0, (CHA + 5 + RING - 1) // RING + 1, step, 0)
        plsc.subcore_barrier()

        @pl.when(s < NS - 1)
        def _():
            pltpu.sync_copy(agg_sh.at[pl.ds(row0, TRB)],
                            agg_out.at[c, pl.ds(row0, TRB)])

        @pl.when(s == NS - 1)
        def _():
            pltpu.sync_copy(agg_sh.at[pl.ds(LAST0, LASTR)],
                            agg_out.at[c, pl.ds(LAST0, LASTR)])

        if with_deg:
            pltpu.sync_copy(ones_hbm, ones_v)
            plsc.subcore_barrier()
            zero_acc()
            plsc.subcore_barrier()
            ssd = (ss[0], ss[1])

            def dload_idx(b, chunk):
                off = pl.multiple_of(base + chunk * K, 8)
                pltpu.sync_copy(dstr_hbm.at[pl.ds(off, K)], ddst_v.at[b])

            def dfire_scatter(b):
                pltpu.async_copy(ones_v, agg_sh.at[ddst_v.at[b]], ssd[b],
                                 add=True)

            def dwait_scatter(b):
                pltpu.make_async_copy(ones_v, agg_sh.at[ddst_v.at[b]],
                                      ssd[b]).wait()

            dload_idx(0, 0)

            def dstep(g2, carry):
                g0 = 2 * g2
                dfire_scatter(0)

                @pl.when(g2 > 0)
                def _():
                    dwait_scatter(1)
                dload_idx(1, g0 + 1)
                dfire_scatter(1)
                dwait_scatter(0)
                dload_idx(0, g0 + 2)
                return carry

            lax.fori_loop(0, (CH - 1) // 2, dstep, 0)
            dfire_scatter(0)
            dwait_scatter(1)
            dwait_scatter(0)
            plsc.subcore_barrier()

            @pl.when(s < NS - 1)
            def _():
                pltpu.sync_copy(agg_sh.at[pl.ds(row0, TRB)],
                                deg_out.at[c, pl.ds(row0, TRB)])

            @pl.when(s == NS - 1)
            def _():
                pltpu.sync_copy(agg_sh.at[pl.ds(LAST0, LASTR)],
                                deg_out.at[c, pl.ds(LAST0, LASTR)])

    out_type = jax.ShapeDtypeStruct((NC, N, D), jnp.float32)
    if with_deg:
        out_type = (out_type, jax.ShapeDtypeStruct((NC, N, D), jnp.float32))
    return pl.kernel(body, out_type=out_type,
                     mesh=mesh, scratch_types=scratch)


_agg_deg = _make_agg_kernel(True)
_agg = _make_agg_kernel(False)


R = 1000


def _tc_layer0_body(feat_ref, agg_ref, deg_ref, ws_ref, wn_ref, b_ref,
                    g_ref, beta_ref, out_ref):
    aggp = agg_ref[...]
    agg = aggp[0] + aggp[1]
    degp = deg_ref[...]
    deg = (degp[0] + degp[1])[:, 0:1]
    inv = 1.0 / jnp.maximum(deg, 1.0)
    mean = agg * inv
    h = (jnp.dot(feat_ref[...], ws_ref[...], preferred_element_type=jnp.float32)
         + jnp.dot(mean, wn_ref[...], preferred_element_type=jnp.float32)
         + b_ref[...])
    mu = jnp.mean(h, axis=-1, keepdims=True)
    var = jnp.mean((h - mu) ** 2, axis=-1, keepdims=True)
    hn = (h - mu) * lax.rsqrt(var + 1e-5) * g_ref[...] + beta_ref[...]
    out_ref[...] = jnp.maximum(hn, 0.0)


def _tc_layer1_body(h_ref, agg_ref, deg_ref, ws_ref, wn_ref, b_ref, out_ref):
    aggp = agg_ref[...]
    agg = aggp[0] + aggp[1]
    degp = deg_ref[...]
    deg = (degp[0] + degp[1])[:, 0:1]
    inv = 1.0 / jnp.maximum(deg, 1.0)
    mean = agg * inv
    out_ref[...] = (
        jnp.dot(h_ref[...], ws_ref[...], preferred_element_type=jnp.float32)
        + jnp.dot(mean, wn_ref[...], preferred_element_type=jnp.float32)
        + b_ref[...])


def _full(shape):
    return pl.BlockSpec(shape, lambda i: (0,) * len(shape))


_row_spec = pl.BlockSpec((R, D), lambda i: (i, 0))
_agg_spec = pl.BlockSpec((NC, R, D), lambda i: (0, i, 0))
_deg_spec = pl.BlockSpec((NC, R, D), lambda i: (0, i, 0))

_tc_layer0 = pl.pallas_call(
    _tc_layer0_body,
    grid=(N // R,),
    in_specs=[_row_spec, _agg_spec, _deg_spec, _full((D, D)), _full((D, D)),
              _full((1, D)), _full((1, D)), _full((1, D))],
    out_specs=_row_spec,
    out_shape=jax.ShapeDtypeStruct((N, D), jnp.float32),
)

_tc_layer1 = pl.pallas_call(
    _tc_layer1_body,
    grid=(N // R,),
    in_specs=[_row_spec, _agg_spec, _deg_spec, _full((D, D)), _full((D, D)),
              _full((1, D))],
    out_specs=_row_spec,
    out_shape=jax.ShapeDtypeStruct((N, D), jnp.float32),
)


def kernel(feat, edge_index, W0_self, W0_neigh, b0, ln_g, ln_b,
           W1_self, W1_neigh, b1):
    src = edge_index[0].astype(jnp.int32)
    dst = edge_index[1].astype(jnp.int32)
    zrows = jnp.zeros((LASTR, D), jnp.float32)
    ones = jnp.ones((K, D), jnp.float32)

    agg0, deg = _agg_deg(feat, src, dst, zrows, ones)
    h1 = _tc_layer0(feat, agg0, deg, W0_self, W0_neigh,
                    b0.reshape(1, D), ln_g.reshape(1, D), ln_b.reshape(1, D))
    agg1 = _agg(h1, src, dst, zrows)
    out = _tc_layer1(h1, agg1, deg, W1_self, W1_neigh, b1.reshape(1, D))
    return out


# speedup vs baseline: 1.0015x; 1.0015x over previous
import functools

import jax
import jax.numpy as jnp
from jax import lax
from jax.experimental import pallas as pl
from jax.experimental.pallas import tpu as pltpu
from jax.experimental.pallas import tpu_sc as plsc

N = 10000
E = 320000
D = 128

NC = 2
NS = 16
NW = NC * NS
EPW = E // NW
K = 80
CH = EPW // K
KA = 40
CHA = EPW // KA
CHD = EPW // KA
RING = 6
TRB = 624
LAST0 = TRB * (NS - 1)
LASTR = N - LAST0


def _make_agg_kernel(with_deg: bool):
    mesh = plsc.VectorSubcoreMesh(core_axis_name="c", subcore_axis_name="s")
    scratch = (
        [pltpu.VMEM((RING, KA), jnp.int32)] +
        [pltpu.VMEM((RING, KA), jnp.int32)] +
        [pltpu.VMEM((RING, KA, D), jnp.float32)] +
        [pltpu.VMEM_SHARED((N, D), jnp.float32)] +
        [pltpu.SemaphoreType.DMA] * (3 * RING)
    )
    if with_deg:
        scratch = scratch + [
            pltpu.VMEM((K, D), jnp.float32),
            pltpu.VMEM((2, K), jnp.int32),
        ]

    def body(h_hbm, srcr_hbm, dstr_hbm, zrows_hbm, *rest):
        if with_deg:
            (ones_hbm, agg_out, deg_out, src_v, dst_v, rows_v, agg_sh,
             *sems) = rest
            ones_v, ddst_v = sems[-2], sems[-1]
            sems = sems[:-2]
        else:
            agg_out, src_v, dst_v, rows_v, agg_sh, *sems = rest
        si = sems[0:RING]
        sg = sems[RING:2 * RING]
        ss = sems[2 * RING:3 * RING]
        c = lax.axis_index("c")
        s = lax.axis_index("s")
        wid = s * NC + c
        row0 = pl.multiple_of(s * TRB, 8)

        def zero_acc():
            @pl.when(s < NS - 1)
            def _():
                pltpu.sync_copy(zrows_hbm.at[pl.ds(0, TRB)],
                                agg_sh.at[pl.ds(row0, TRB)])

            @pl.when(s == NS - 1)
            def _():
                pltpu.sync_copy(zrows_hbm.at[pl.ds(0, LASTR)],
                                agg_sh.at[pl.ds(LAST0, LASTR)])

        zero_acc()
        plsc.subcore_barrier()

        base = wid * EPW

        def fire_idx(b, chunk):
            off = pl.multiple_of(base + chunk * KA, 8)
            pltpu.async_copy(srcr_hbm.at[pl.ds(off, KA)], src_v.at[b], si[b])
            pltpu.async_copy(dstr_hbm.at[pl.ds(off, KA)], dst_v.at[b], si[b])

        def wait_idx(b):
            pltpu.make_async_copy(srcr_hbm.at[pl.ds(0, KA)], src_v.at[b],
                                  si[b]).wait()
            pltpu.make_async_copy(dstr_hbm.at[pl.ds(0, KA)], dst_v.at[b],
                                  si[b]).wait()

        def fire_gather(b):
            pltpu.async_copy(h_hbm.at[src_v.at[b]], rows_v.at[b], sg[b])

        def wait_gather(b):
            pltpu.make_async_copy(h_hbm.at[src_v.at[b]], rows_v.at[b],
                                  sg[b]).wait()

        def fire_scatter(b):
            pltpu.async_copy(rows_v.at[b], agg_sh.at[dst_v.at[b]], ss[b],
                             add=True)

        def wait_scatter(b):
            pltpu.make_async_copy(rows_v.at[b], agg_sh.at[dst_v.at[b]],
                                  ss[b]).wait()

        def step(i, carry):
            for u in range(RING):
                t = RING * i + u
                @pl.when(jnp.logical_and(t >= 5, t < CHA + 5))
                def _(u=u):
                    wait_scatter((u - 5) % RING)

                @pl.when(jnp.logical_and(t >= 4, t < CHA + 4))
                def _(u=u):
                    wait_gather((u - 4) % RING)
                    fire_scatter((u - 4) % RING)

                @pl.when(jnp.logical_and(t >= 1, t < CHA + 1))
                def _(u=u):
                    wait_idx((u - 1) % RING)
                    fire_gather((u - 1) % RING)

                @pl.when(t < CHA)
                def _(u=u, t=t):
                    fire_idx(u, t)
            return carry

        lax.fori_loop(0, (CHA + 5 + RING - 1) // RING + 1, step, 0)
        plsc.subcore_barrier()

        @pl.when(s < NS - 1)
        def _():
            pltpu.sync_copy(agg_sh.at[pl.ds(row0, TRB)],
                            agg_out.at[c, pl.ds(row0, TRB)])

        @pl.when(s == NS - 1)
        def _():
            pltpu.sync_copy(agg_sh.at[pl.ds(LAST0, LASTR)],
                            agg_out.at[c, pl.ds(LAST0, LASTR)])

        if with_deg:
            pltpu.sync_copy(ones_hbm, ones_v)
            plsc.subcore_barrier()
            zero_acc()
            plsc.subcore_barrier()
            ssd = (ss[0], ss[1])

            def dload_idx(b, chunk):
                off = pl.multiple_of(base + chunk * K, 8)
                pltpu.sync_copy(dstr_hbm.at[pl.ds(off, K)], ddst_v.at[b])

            def dfire_scatter(b):
                pltpu.async_copy(ones_v, agg_sh.at[ddst_v.at[b]], ssd[b],
                                 add=True)

            def dwait_scatter(b):
                pltpu.make_async_copy(ones_v, agg_sh.at[ddst_v.at[b]],
                                      ssd[b]).wait()

            dload_idx(0, 0)

            def dstep(g2, carry):
                g0 = 2 * g2
                dfire_scatter(0)

                @pl.when(g2 > 0)
                def _():
                    dwait_scatter(1)
                dload_idx(1, g0 + 1)
                dfire_scatter(1)
                dwait_scatter(0)
                dload_idx(0, g0 + 2)
                return carry

            lax.fori_loop(0, (CH - 1) // 2, dstep, 0)
            dfire_scatter(0)
            dwait_scatter(1)
            dwait_scatter(0)
            plsc.subcore_barrier()

            @pl.when(s < NS - 1)
            def _():
                pltpu.sync_copy(agg_sh.at[pl.ds(row0, TRB)],
                                deg_out.at[c, pl.ds(row0, TRB)])

            @pl.when(s == NS - 1)
            def _():
                pltpu.sync_copy(agg_sh.at[pl.ds(LAST0, LASTR)],
                                deg_out.at[c, pl.ds(LAST0, LASTR)])

    out_type = jax.ShapeDtypeStruct((NC, N, D), jnp.float32)
    if with_deg:
        out_type = (out_type, jax.ShapeDtypeStruct((NC, N, D), jnp.float32))
    return pl.kernel(body, out_type=out_type,
                     mesh=mesh, scratch_types=scratch)


_agg_deg = _make_agg_kernel(True)
_agg = _make_agg_kernel(False)


R = 1000


def _tc_layer0_body(feat_ref, agg_ref, deg_ref, ws_ref, wn_ref, b_ref,
                    g_ref, beta_ref, out_ref):
    aggp = agg_ref[...]
    agg = aggp[0] + aggp[1]
    degp = deg_ref[...]
    deg = (degp[0] + degp[1])[:, 0:1]
    inv = 1.0 / jnp.maximum(deg, 1.0)
    mean = agg * inv
    h = (jnp.dot(feat_ref[...], ws_ref[...], preferred_element_type=jnp.float32)
         + jnp.dot(mean, wn_ref[...], preferred_element_type=jnp.float32)
         + b_ref[...])
    mu = jnp.mean(h, axis=-1, keepdims=True)
    var = jnp.mean((h - mu) ** 2, axis=-1, keepdims=True)
    hn = (h - mu) * lax.rsqrt(var + 1e-5) * g_ref[...] + beta_ref[...]
    out_ref[...] = jnp.maximum(hn, 0.0)


def _tc_layer1_body(h_ref, agg_ref, deg_ref, ws_ref, wn_ref, b_ref, out_ref):
    aggp = agg_ref[...]
    agg = aggp[0] + aggp[1]
    degp = deg_ref[...]
    deg = (degp[0] + degp[1])[:, 0:1]
    inv = 1.0 / jnp.maximum(deg, 1.0)
    mean = agg * inv
    out_ref[...] = (
        jnp.dot(h_ref[...], ws_ref[...], preferred_element_type=jnp.float32)
        + jnp.dot(mean, wn_ref[...], preferred_element_type=jnp.float32)
        + b_ref[...])


def _full(shape):
    return pl.BlockSpec(shape, lambda i: (0,) * len(shape))


_row_spec = pl.BlockSpec((R, D), lambda i: (i, 0))
_agg_spec = pl.BlockSpec((NC, R, D), lambda i: (0, i, 0))
_deg_spec = pl.BlockSpec((NC, R, D), lambda i: (0, i, 0))

_tc_layer0 = pl.pallas_call(
    _tc_layer0_body,
    grid=(N // R,),
    in_specs=[_row_spec, _agg_spec, _deg_spec, _full((D, D)), _full((D, D)),
              _full((1, D)), _full((1, D)), _full((1, D))],
    out_specs=_row_spec,
    out_shape=jax.ShapeDtypeStruct((N, D), jnp.float32),
)

_tc_layer1 = pl.pallas_call(
    _tc_layer1_body,
    grid=(N // R,),
    in_specs=[_row_spec, _agg_spec, _deg_spec, _full((D, D)), _full((D, D)),
              _full((1, D))],
    out_specs=_row_spec,
    out_shape=jax.ShapeDtypeStruct((N, D), jnp.float32),
)


def kernel(feat, edge_index, W0_self, W0_neigh, b0, ln_g, ln_b,
           W1_self, W1_neigh, b1):
    src = edge_index[0].astype(jnp.int32)
    dst = edge_index[1].astype(jnp.int32)
    zrows = jnp.zeros((LASTR, D), jnp.float32)
    ones = jnp.ones((K, D), jnp.float32)

    agg0, deg = _agg_deg(feat, src, dst, zrows, ones)
    h1 = _tc_layer0(feat, agg0, deg, W0_self, W0_neigh,
                    b0.reshape(1, D), ln_g.reshape(1, D), ln_b.reshape(1, D))
    agg1 = _agg(h1, src, dst, zrows)
    out = _tc_layer1(h1, agg1, deg, W1_self, W1_neigh, b1.reshape(1, D))
    return out


# speedup vs baseline: 1.0016x; 1.0001x over previous
import jax
import jax.numpy as jnp
from jax import lax
from jax.experimental import pallas as pl
from jax.experimental.pallas import tpu as pltpu
from jax.experimental.pallas import tpu_sc as plsc

N = 10000
E = 320000
D = 128

NC = 2
NS = 16
NW = NC * NS
EPW = E // NW
K = 80
CH = EPW // K
KA = 40
CHA = EPW // KA
CHD = EPW // KA
RING = 6
TRB = 624
LAST0 = TRB * (NS - 1)
LASTR = N - LAST0


def _make_agg_kernel(with_deg: bool):
    mesh = plsc.VectorSubcoreMesh(core_axis_name="c", subcore_axis_name="s")
    scratch = (
        [pltpu.VMEM((RING, KA), jnp.int32)] +
        [pltpu.VMEM((RING, KA), jnp.int32)] +
        [pltpu.VMEM((RING, KA, D), jnp.float32)] +
        [pltpu.VMEM_SHARED((N, D), jnp.float32)] +
        [pltpu.SemaphoreType.DMA] * (3 * RING)
    )
    if with_deg:
        scratch = scratch + [
            pltpu.VMEM((K, D), jnp.float32),
            pltpu.VMEM((2, K), jnp.int32),
        ]

    def body(h_hbm, srcr_hbm, dstr_hbm, zrows_hbm, *rest):
        if with_deg:
            (ones_hbm, agg_out, deg_out, src_v, dst_v, rows_v, agg_sh,
             *sems) = rest
            ones_v, ddst_v = sems[-2], sems[-1]
            sems = sems[:-2]
        else:
            agg_out, src_v, dst_v, rows_v, agg_sh, *sems = rest
        si = sems[0:RING]
        sg = sems[RING:2 * RING]
        ss = sems[2 * RING:3 * RING]
        c = lax.axis_index("c")
        s = lax.axis_index("s")
        wid = s * NC + c
        row0 = pl.multiple_of(s * TRB, 8)

        def zero_acc():
            @pl.when(s < NS - 1)
            def _():
                pltpu.sync_copy(zrows_hbm.at[pl.ds(0, TRB)],
                                agg_sh.at[pl.ds(row0, TRB)])

            @pl.when(s == NS - 1)
            def _():
                pltpu.sync_copy(zrows_hbm.at[pl.ds(0, LASTR)],
                                agg_sh.at[pl.ds(LAST0, LASTR)])

        zero_acc()
        plsc.subcore_barrier()

        base = wid * EPW

        def fire_idx(b, chunk):
            off = pl.multiple_of(base + chunk * KA, 8)
            pltpu.async_copy(srcr_hbm.at[pl.ds(off, KA)], src_v.at[b], si[b])
            pltpu.async_copy(dstr_hbm.at[pl.ds(off, KA)], dst_v.at[b], si[b])

        def wait_idx(b):
            pltpu.make_async_copy(srcr_hbm.at[pl.ds(0, KA)], src_v.at[b],
                                  si[b]).wait()
            pltpu.make_async_copy(dstr_hbm.at[pl.ds(0, KA)], dst_v.at[b],
                                  si[b]).wait()

        def fire_gather(b):
            pltpu.async_copy(h_hbm.at[src_v.at[b]], rows_v.at[b], sg[b])

        def wait_gather(b):
            pltpu.make_async_copy(h_hbm.at[src_v.at[b]], rows_v.at[b],
                                  sg[b]).wait()

        def fire_scatter(b):
            pltpu.async_copy(rows_v.at[b], agg_sh.at[dst_v.at[b]], ss[b],
                             add=True)

        def wait_scatter(b):
            pltpu.make_async_copy(rows_v.at[b], agg_sh.at[dst_v.at[b]],
                                  ss[b]).wait()

        def step(i, carry):
            for u in range(RING):
                t = RING * i + u
                @pl.when(jnp.logical_and(t >= 5, t < CHA + 5))
                def _(u=u):
                    wait_scatter((u - 5) % RING)

                @pl.when(jnp.logical_and(t >= 4, t < CHA + 4))
                def _(u=u):
                    wait_gather((u - 4) % RING)
                    fire_scatter((u - 4) % RING)

                @pl.when(jnp.logical_and(t >= 1, t < CHA + 1))
                def _(u=u):
                    wait_idx((u - 1) % RING)
                    fire_gather((u - 1) % RING)

                @pl.when(t < CHA)
                def _(u=u, t=t):
                    fire_idx(u, t)
            return carry

        lax.fori_loop(0, (CHA + 5 + RING - 1) // RING + 1, step, 0)
        plsc.subcore_barrier()

        @pl.when(s < NS - 1)
        def _():
            pltpu.sync_copy(agg_sh.at[pl.ds(row0, TRB)],
                            agg_out.at[c, pl.ds(row0, TRB)])

        @pl.when(s == NS - 1)
        def _():
            pltpu.sync_copy(agg_sh.at[pl.ds(LAST0, LASTR)],
                            agg_out.at[c, pl.ds(LAST0, LASTR)])

        if with_deg:
            pltpu.sync_copy(ones_hbm, ones_v)
            plsc.subcore_barrier()
            zero_acc()
            plsc.subcore_barrier()
            ssd = (ss[0], ss[1])

            def dload_idx(b, chunk):
                off = pl.multiple_of(base + chunk * K, 8)
                pltpu.sync_copy(dstr_hbm.at[pl.ds(off, K)], ddst_v.at[b])

            def dfire_scatter(b):
                pltpu.async_copy(ones_v, agg_sh.at[ddst_v.at[b]], ssd[b],
                                 add=True)

            def dwait_scatter(b):
                pltpu.make_async_copy(ones_v, agg_sh.at[ddst_v.at[b]],
                                      ssd[b]).wait()

            dload_idx(0, 0)

            def dstep(g2, carry):
                g0 = 2 * g2
                dfire_scatter(0)

                @pl.when(g2 > 0)
                def _():
                    dwait_scatter(1)
                dload_idx(1, g0 + 1)
                dfire_scatter(1)
                dwait_scatter(0)
                dload_idx(0, g0 + 2)
                return carry

            lax.fori_loop(0, (CH - 1) // 2, dstep, 0)
            dfire_scatter(0)
            dwait_scatter(1)
            dwait_scatter(0)
            plsc.subcore_barrier()

            @pl.when(s < NS - 1)
            def _():
                pltpu.sync_copy(agg_sh.at[pl.ds(row0, TRB)],
                                deg_out.at[c, pl.ds(row0, TRB)])

            @pl.when(s == NS - 1)
            def _():
                pltpu.sync_copy(agg_sh.at[pl.ds(LAST0, LASTR)],
                                deg_out.at[c, pl.ds(LAST0, LASTR)])

    out_type = jax.ShapeDtypeStruct((NC, N, D), jnp.float32)
    if with_deg:
        out_type = (out_type, jax.ShapeDtypeStruct((NC, N, D), jnp.float32))
    return pl.kernel(body, out_type=out_type,
                     mesh=mesh, scratch_types=scratch)


_agg_deg = _make_agg_kernel(True)
_agg = _make_agg_kernel(False)


R = 1000


def _tc_layer0_body(feat_ref, agg_ref, deg_ref, ws_ref, wn_ref, b_ref,
                    g_ref, beta_ref, out_ref):
    aggp = agg_ref[...]
    agg = aggp[0] + aggp[1]
    degp = deg_ref[...]
    deg = (degp[0] + degp[1])[:, 0:1]
    inv = 1.0 / jnp.maximum(deg, 1.0)
    mean = agg * inv
    h = (jnp.dot(feat_ref[...], ws_ref[...], preferred_element_type=jnp.float32)
         + jnp.dot(mean, wn_ref[...], preferred_element_type=jnp.float32)
         + b_ref[...])
    mu = jnp.mean(h, axis=-1, keepdims=True)
    var = jnp.mean((h - mu) ** 2, axis=-1, keepdims=True)
    hn = (h - mu) * lax.rsqrt(var + 1e-5) * g_ref[...] + beta_ref[...]
    out_ref[...] = jnp.maximum(hn, 0.0)


def _tc_layer1_body(h_ref, agg_ref, deg_ref, ws_ref, wn_ref, b_ref, out_ref):
    aggp = agg_ref[...]
    agg = aggp[0] + aggp[1]
    degp = deg_ref[...]
    deg = (degp[0] + degp[1])[:, 0:1]
    inv = 1.0 / jnp.maximum(deg, 1.0)
    mean = agg * inv
    out_ref[...] = (
        jnp.dot(h_ref[...], ws_ref[...], preferred_element_type=jnp.float32)
        + jnp.dot(mean, wn_ref[...], preferred_element_type=jnp.float32)
        + b_ref[...])


def _full(shape):
    return pl.BlockSpec(shape, lambda i: (0,) * len(shape))


_row_spec = pl.BlockSpec((R, D), lambda i: (i, 0))
_agg_spec = pl.BlockSpec((NC, R, D), lambda i: (0, i, 0))
_deg_spec = pl.BlockSpec((NC, R, D), lambda i: (0, i, 0))

_tc_layer0 = pl.pallas_call(
    _tc_layer0_body,
    grid=(N // R,),
    in_specs=[_row_spec, _agg_spec, _deg_spec, _full((D, D)), _full((D, D)),
              _full((1, D)), _full((1, D)), _full((1, D))],
    out_specs=_row_spec,
    out_shape=jax.ShapeDtypeStruct((N, D), jnp.float32),
)

_tc_layer1 = pl.pallas_call(
    _tc_layer1_body,
    grid=(N // R,),
    in_specs=[_row_spec, _agg_spec, _deg_spec, _full((D, D)), _full((D, D)),
              _full((1, D))],
    out_specs=_row_spec,
    out_shape=jax.ShapeDtypeStruct((N, D), jnp.float32),
)


def kernel(feat, edge_index, W0_self, W0_neigh, b0, ln_g, ln_b,
           W1_self, W1_neigh, b1):
    src = edge_index[0].astype(jnp.int32)
    dst = edge_index[1].astype(jnp.int32)
    zrows = jnp.zeros((LASTR, D), jnp.float32)
    ones = jnp.ones((K, D), jnp.float32)

    agg0, deg = _agg_deg(feat, src, dst, zrows, ones)
    h1 = _tc_layer0(feat, agg0, deg, W0_self, W0_neigh,
                    b0.reshape(1, D), ln_g.reshape(1, D), ln_b.reshape(1, D))
    agg1 = _agg(h1, src, dst, zrows)
    out = _tc_layer1(h1, agg1, deg, W1_self, W1_neigh, b1.reshape(1, D))
    return out
